# Initial kernel scaffold; baseline (speedup 1.0000x reference)
import jax
import jax.numpy as jnp
from jax.experimental import pallas as pl


def kernel(relative_pos, edge_index_i, kernel_dirs, W, b, ln_gamma, ln_beta):
    raise NotImplementedError("write your pallas kernel here")


# speedup vs baseline: 33.3554x; 33.3554x over previous
import functools

import jax
import jax.numpy as jnp
from jax import lax
from jax.experimental import pallas as pl
from jax.experimental.pallas import tpu as pltpu
from jax.experimental.pallas import tpu_sc as plsc

N_NODES = 100000
N_EDGES = 6400000
NC = 2
NS = 16
NW = NC * NS
CH = 128
NCH = 16
BLK = NCH * CH
NBLKS = N_EDGES // BLK
KMAX = -(-NBLKS // NW)
NROW = N_EDGES // CH
N_PAD = NW * 3136
NODES_W = N_PAD // NW
NODES_S = N_PAD // NS
RW = 8


def _rsqrt(x):
    i = lax.bitcast_convert_type(x, jnp.int32)
    i = jnp.int32(0x5F3759DF) - lax.shift_right_arithmetic(i, 1)
    y = lax.bitcast_convert_type(i, jnp.float32)
    y = y * (1.5 - 0.5 * x * y * y)
    y = y * (1.5 - 0.5 * x * y * y)
    return y


def _bcast(vec_ref, k):
    assert k >= 1
    return plsc.load_gather(vec_ref, [jnp.full((16,), k, jnp.int32)])


_MESH = plsc.VectorSubcoreMesh(core_axis_name="c", subcore_axis_name="s",
                               num_cores=NC, num_subcores=NS)
_CP = pltpu.CompilerParams(needs_layout_passes=False, use_tc_tiling_on_sc=False)


@functools.partial(
    pl.kernel,
    compiler_params=_CP,
    out_type=jax.ShapeDtypeStruct((NC, N_PAD, RW), jnp.float32),
    mesh=_MESH,
    scratch_types=[
        pltpu.VMEM((BLK,), jnp.float32),
        pltpu.VMEM((BLK,), jnp.float32),
        pltpu.VMEM((BLK,), jnp.float32),
        pltpu.VMEM((NCH, CH), jnp.int32),
        pltpu.VMEM((BLK, RW), jnp.float32),
        pltpu.VMEM((16,), jnp.float32),
        pltpu.VMEM_SHARED((N_PAD, RW), jnp.float32),
        pltpu.SemaphoreType.DMA,
    ],
)
def _scatter_pass(px_hbm, py_hbm, pz_hbm, idx_hbm, zeros_hbm, m3_hbm,
                  acc_hbm, px_v, py_v, pz_v, idx_v, contrib, m_v, acc_sh, sem):
    c = lax.axis_index("c")
    s = lax.axis_index("s")
    w = c * NS + s

    pltpu.sync_copy(zeros_hbm.at[pl.ds(NODES_S * s, NODES_S), :],
                    acc_sh.at[pl.ds(NODES_S * s, NODES_S), :])
    pltpu.sync_copy(m3_hbm, m_v)
    pltpu.sync_copy(zeros_hbm.at[pl.ds(0, BLK), :], contrib)
    plsc.subcore_barrier()

    lane = lax.iota(jnp.int32, 16)
    m00 = _bcast(m_v, 1); m01 = _bcast(m_v, 2); m02 = _bcast(m_v, 3)
    m10 = _bcast(m_v, 4); m11 = _bcast(m_v, 5); m12 = _bcast(m_v, 6)
    m20 = _bcast(m_v, 7); m21 = _bcast(m_v, 8); m22 = _bcast(m_v, 9)
    col0 = jnp.full((16,), 0, jnp.int32)
    col1 = jnp.full((16,), 1, jnp.int32)
    col2 = jnp.full((16,), 2, jnp.int32)
    col3 = jnp.full((16,), 3, jnp.int32)
    ones = jnp.full((16,), 1.0, jnp.float32)

    def block_body(k, carry):
        blk = k * NW + w

        @pl.when(blk < NBLKS)
        def _():
            eb = pl.multiple_of(blk * BLK, 8)
            pltpu.sync_copy(px_hbm.at[pl.ds(eb, BLK)], px_v)
            pltpu.sync_copy(py_hbm.at[pl.ds(eb, BLK)], py_v)
            pltpu.sync_copy(pz_hbm.at[pl.ds(eb, BLK)], pz_v)
            pltpu.sync_copy(
                idx_hbm.at[pl.ds(pl.multiple_of(blk * NCH, 8), NCH), :], idx_v)

            def grp(g, carry2):
                o = g * 16
                px = px_v[pl.ds(o, 16)]
                py = py_v[pl.ds(o, 16)]
                pz = pz_v[pl.ds(o, 16)]
                n2 = px * px + py * py + pz * pz
                r = _rsqrt(n2)
                inv = 1.0 / (n2 * r + 1e-8)
                dx = px * inv
                dy = py * inv
                dz = pz * inv
                cx = dx * m00 + dy * m10 + dz * m20
                cy = dx * m01 + dy * m11 + dz * m21
                cz = dx * m02 + dy * m12 + dz * m22
                p16 = o + lane
                plsc.store_scatter(contrib, [p16, col0], cx)
                plsc.store_scatter(contrib, [p16, col1], cy)
                plsc.store_scatter(contrib, [p16, col2], cz)
                plsc.store_scatter(contrib, [p16, col3], ones)
                return carry2

            lax.fori_loop(0, BLK // 16, grp, 0)

            copies = [
                pltpu.async_copy(contrib.at[pl.ds(CH * j, CH), :],
                                 acc_sh.at[idx_v.at[j]], sem, add=True)
                for j in range(NCH)
            ]
            for d in copies:
                d.wait()

        return carry

    lax.fori_loop(0, KMAX, block_body, 0)
    plsc.subcore_barrier()
    pltpu.sync_copy(acc_sh.at[pl.ds(NODES_S * s, NODES_S), :],
                    acc_hbm.at[c, pl.ds(NODES_S * s, NODES_S), :])


@functools.partial(
    pl.kernel,
    compiler_params=_CP,
    out_type=jax.ShapeDtypeStruct((N_PAD, RW), jnp.float32),
    mesh=_MESH,
    scratch_types=[
        pltpu.VMEM((NODES_W, RW), jnp.float32),
        pltpu.VMEM((NODES_W, RW), jnp.float32),
        pltpu.VMEM((NODES_W, RW), jnp.float32),
        pltpu.VMEM((16,), jnp.float32),
    ],
)
def _node_pass(acc_hbm, par_hbm, feat_hbm, a0, a1, fo, par_v):
    c = lax.axis_index("c")
    s = lax.axis_index("s")
    w = c * NS + s
    nb = w * NODES_W
    pltpu.sync_copy(acc_hbm.at[0, pl.ds(nb, NODES_W), :], a0)
    pltpu.sync_copy(acc_hbm.at[1, pl.ds(nb, NODES_W), :], a1)
    pltpu.sync_copy(par_hbm, par_v)

    lane = lax.iota(jnp.int32, 16)
    b0 = _bcast(par_v, 1); b1 = _bcast(par_v, 2); b2 = _bcast(par_v, 3)
    g0 = _bcast(par_v, 4); g1 = _bcast(par_v, 5); g2 = _bcast(par_v, 6)
    e0 = _bcast(par_v, 7); e1 = _bcast(par_v, 8); e2 = _bcast(par_v, 9)
    col0 = jnp.full((16,), 0, jnp.int32)
    col1 = jnp.full((16,), 1, jnp.int32)
    col2 = jnp.full((16,), 2, jnp.int32)
    col3 = jnp.full((16,), 3, jnp.int32)
    zf = jnp.zeros((16,), jnp.float32)

    def grp(g, carry):
        row = g * 16 + lane
        xs = plsc.load_gather(a0, [row, col0]) + plsc.load_gather(a1, [row, col0])
        ys = plsc.load_gather(a0, [row, col1]) + plsc.load_gather(a1, [row, col1])
        zs = plsc.load_gather(a0, [row, col2]) + plsc.load_gather(a1, [row, col2])
        cn = plsc.load_gather(a0, [row, col3]) + plsc.load_gather(a1, [row, col3])
        cnt = jnp.maximum(cn, 1.0)
        fx = xs / cnt + b0
        fy = ys / cnt + b1
        fz = zs / cnt + b2
        mu = (fx + fy + fz) * jnp.float32(1.0 / 3.0)
        ex = fx - mu
        ey = fy - mu
        ez = fz - mu
        var = (ex * ex + ey * ey + ez * ez) * jnp.float32(1.0 / 3.0)
        rs = _rsqrt(var + 1e-5)
        plsc.store_scatter(fo, [row, col0], ex * rs * g0 + e0)
        plsc.store_scatter(fo, [row, col1], ey * rs * g1 + e1)
        plsc.store_scatter(fo, [row, col2], ez * rs * g2 + e2)
        plsc.store_scatter(fo, [row, col3], zf)
        return carry

    lax.fori_loop(0, NODES_W // 16, grp, 0)
    pltpu.sync_copy(fo, feat_hbm.at[pl.ds(nb, NODES_W), :])


@functools.partial(
    pl.kernel,
    compiler_params=_CP,
    out_type=[jax.ShapeDtypeStruct((N_EDGES,), jnp.float32)] * 3,
    mesh=_MESH,
    scratch_types=[
        pltpu.VMEM((NCH, CH), jnp.int32),
        pltpu.VMEM((BLK, RW), jnp.float32),
        pltpu.VMEM((BLK,), jnp.float32),
        pltpu.VMEM((BLK,), jnp.float32),
        pltpu.VMEM((BLK,), jnp.float32),
        pltpu.SemaphoreType.DMA,
    ],
)
def _gather_pass(feat_hbm, idx_hbm, ox_hbm, oy_hbm, oz_hbm,
                 idx_v, rows_v, ox_v, oy_v, oz_v, sem):
    c = lax.axis_index("c")
    s = lax.axis_index("s")
    w = c * NS + s
    lane = lax.iota(jnp.int32, 16)
    col0 = jnp.full((16,), 0, jnp.int32)
    col1 = jnp.full((16,), 1, jnp.int32)
    col2 = jnp.full((16,), 2, jnp.int32)

    def block_body(k, carry):
        blk = k * NW + w

        @pl.when(blk < NBLKS)
        def _():
            eb = pl.multiple_of(blk * BLK, 8)
            pltpu.sync_copy(
                idx_hbm.at[pl.ds(pl.multiple_of(blk * NCH, 8), NCH), :], idx_v)
            copies = [
                pltpu.async_copy(feat_hbm.at[idx_v.at[j]],
                                 rows_v.at[pl.ds(CH * j, CH), :], sem)
                for j in range(NCH)
            ]
            for d in copies:
                d.wait()

            def grp(g, carry2):
                o = g * 16
                row = o + lane
                ox_v[pl.ds(o, 16)] = plsc.load_gather(rows_v, [row, col0])
                oy_v[pl.ds(o, 16)] = plsc.load_gather(rows_v, [row, col1])
                oz_v[pl.ds(o, 16)] = plsc.load_gather(rows_v, [row, col2])
                return carry2

            lax.fori_loop(0, BLK // 16, grp, 0)
            pltpu.sync_copy(ox_v, ox_hbm.at[pl.ds(eb, BLK)])
            pltpu.sync_copy(oy_v, oy_hbm.at[pl.ds(eb, BLK)])
            pltpu.sync_copy(oz_v, oz_hbm.at[pl.ds(eb, BLK)])

        return carry

    lax.fori_loop(0, KMAX, block_body, 0)


def kernel(relative_pos, edge_index_i, kernel_dirs, W, b, ln_gamma, ln_beta):
    m3 = kernel_dirs.T @ W.T
    pad1 = jnp.zeros((1,), jnp.float32)
    m3_pad = jnp.concatenate([pad1, m3.reshape(9), jnp.zeros((6,), jnp.float32)])
    params = jnp.concatenate(
        [pad1, b, ln_gamma, ln_beta, jnp.zeros((6,), jnp.float32)])

    px = relative_pos[:, 0]
    py = relative_pos[:, 1]
    pz = relative_pos[:, 2]
    idx2d = edge_index_i.reshape(NROW, CH)
    zeros = jnp.zeros((N_PAD, RW), jnp.float32)

    acc = _scatter_pass(px, py, pz, idx2d, zeros, m3_pad)
    feat = _node_pass(acc, params)
    ox, oy, oz = _gather_pass(feat, idx2d)
    return jnp.stack([ox, oy, oz], axis=1)


# speedup vs baseline: 37.3382x; 1.1194x over previous
import functools

import jax
import jax.numpy as jnp
from jax import lax
from jax.experimental import pallas as pl
from jax.experimental.pallas import tpu as pltpu
from jax.experimental.pallas import tpu_sc as plsc

N_NODES = 100000
N_EDGES = 6400000
NC = 2
NS = 16
NW = NC * NS
CH = 128
NCH = 16
BLK = NCH * CH
NBLKS = N_EDGES // BLK
KMAX = -(-NBLKS // NW)
NROW = N_EDGES // CH
N_PAD = NW * 3136
NODES_W = N_PAD // NW
NODES_S = N_PAD // NS
RW = 8


def _rsqrt(x):
    i = lax.bitcast_convert_type(x, jnp.int32)
    i = jnp.int32(0x5F3759DF) - lax.shift_right_arithmetic(i, 1)
    y = lax.bitcast_convert_type(i, jnp.float32)
    y = y * (1.5 - 0.5 * x * y * y)
    y = y * (1.5 - 0.5 * x * y * y)
    return y


def _bcast(vec_ref, k):
    assert k >= 1
    return plsc.load_gather(vec_ref, [jnp.full((16,), k, jnp.int32)])


_MESH = plsc.VectorSubcoreMesh(core_axis_name="c", subcore_axis_name="s",
                               num_cores=NC, num_subcores=NS)
_CP = pltpu.CompilerParams(needs_layout_passes=False, use_tc_tiling_on_sc=False)


@functools.partial(
    pl.kernel,
    compiler_params=_CP,
    out_type=jax.ShapeDtypeStruct((NC, N_PAD, RW), jnp.float32),
    mesh=_MESH,
    scratch_types=[
        pltpu.VMEM((BLK,), jnp.float32),
        pltpu.VMEM((BLK,), jnp.float32),
        pltpu.VMEM((BLK,), jnp.float32),
        pltpu.VMEM((NCH, CH), jnp.int32),
        pltpu.VMEM((BLK, RW), jnp.float32),
        pltpu.VMEM((16,), jnp.float32),
        pltpu.VMEM_SHARED((N_PAD, RW), jnp.float32),
        pltpu.SemaphoreType.DMA,
        pltpu.SemaphoreType.DMA,
    ],
)
def _scatter_pass(px_hbm, py_hbm, pz_hbm, idx_hbm, zeros_hbm, m3_hbm,
                  acc_hbm, px_v, py_v, pz_v, idx_v, contrib, m_v, acc_sh, sem,
                  sem_in):
    c = lax.axis_index("c")
    s = lax.axis_index("s")
    w = c * NS + s

    pltpu.sync_copy(zeros_hbm.at[pl.ds(NODES_S * s, NODES_S), :],
                    acc_sh.at[pl.ds(NODES_S * s, NODES_S), :])
    pltpu.sync_copy(m3_hbm, m_v)
    pltpu.sync_copy(zeros_hbm.at[pl.ds(0, BLK), :], contrib)
    plsc.subcore_barrier()

    lane = lax.iota(jnp.int32, 16)
    m00 = _bcast(m_v, 1); m01 = _bcast(m_v, 2); m02 = _bcast(m_v, 3)
    m10 = _bcast(m_v, 4); m11 = _bcast(m_v, 5); m12 = _bcast(m_v, 6)
    m20 = _bcast(m_v, 7); m21 = _bcast(m_v, 8); m22 = _bcast(m_v, 9)
    col0 = jnp.full((16,), 0, jnp.int32)
    col1 = jnp.full((16,), 1, jnp.int32)
    col2 = jnp.full((16,), 2, jnp.int32)
    col3 = jnp.full((16,), 3, jnp.int32)
    ones = jnp.full((16,), 1.0, jnp.float32)

    def block_body(k, carry):
        blk = k * NW + w

        @pl.when(blk < NBLKS)
        def _():
            eb = pl.multiple_of(blk * BLK, 8)
            loads = [
                pltpu.async_copy(px_hbm.at[pl.ds(eb, BLK)], px_v, sem_in),
                pltpu.async_copy(py_hbm.at[pl.ds(eb, BLK)], py_v, sem_in),
                pltpu.async_copy(pz_hbm.at[pl.ds(eb, BLK)], pz_v, sem_in),
                pltpu.async_copy(
                    idx_hbm.at[pl.ds(pl.multiple_of(blk * NCH, 8), NCH), :],
                    idx_v, sem_in),
            ]
            for d in loads:
                d.wait()

            def grp(g, carry2):
                o = g * 16
                px = px_v[pl.ds(o, 16)]
                py = py_v[pl.ds(o, 16)]
                pz = pz_v[pl.ds(o, 16)]
                n2 = px * px + py * py + pz * pz
                r = _rsqrt(n2)
                inv = 1.0 / (n2 * r + 1e-8)
                dx = px * inv
                dy = py * inv
                dz = pz * inv
                cx = dx * m00 + dy * m10 + dz * m20
                cy = dx * m01 + dy * m11 + dz * m21
                cz = dx * m02 + dy * m12 + dz * m22
                p16 = o + lane
                plsc.store_scatter(contrib, [p16, col0], cx)
                plsc.store_scatter(contrib, [p16, col1], cy)
                plsc.store_scatter(contrib, [p16, col2], cz)
                plsc.store_scatter(contrib, [p16, col3], ones)
                return carry2

            lax.fori_loop(0, BLK // 16, grp, 0)

            copies = [
                pltpu.async_copy(contrib.at[pl.ds(CH * j, CH), :],
                                 acc_sh.at[idx_v.at[j]], sem, add=True)
                for j in range(NCH)
            ]
            for d in copies:
                d.wait()

        return carry

    lax.fori_loop(0, KMAX, block_body, 0)
    plsc.subcore_barrier()
    pltpu.sync_copy(acc_sh.at[pl.ds(NODES_S * s, NODES_S), :],
                    acc_hbm.at[c, pl.ds(NODES_S * s, NODES_S), :])


@functools.partial(
    pl.kernel,
    compiler_params=_CP,
    out_type=jax.ShapeDtypeStruct((N_PAD, RW), jnp.float32),
    mesh=_MESH,
    scratch_types=[
        pltpu.VMEM((NODES_W, RW), jnp.float32),
        pltpu.VMEM((NODES_W, RW), jnp.float32),
        pltpu.VMEM((NODES_W, RW), jnp.float32),
        pltpu.VMEM((16,), jnp.float32),
    ],
)
def _node_pass(acc_hbm, par_hbm, feat_hbm, a0, a1, fo, par_v):
    c = lax.axis_index("c")
    s = lax.axis_index("s")
    w = c * NS + s
    nb = w * NODES_W
    pltpu.sync_copy(acc_hbm.at[0, pl.ds(nb, NODES_W), :], a0)
    pltpu.sync_copy(acc_hbm.at[1, pl.ds(nb, NODES_W), :], a1)
    pltpu.sync_copy(par_hbm, par_v)

    lane = lax.iota(jnp.int32, 16)
    b0 = _bcast(par_v, 1); b1 = _bcast(par_v, 2); b2 = _bcast(par_v, 3)
    g0 = _bcast(par_v, 4); g1 = _bcast(par_v, 5); g2 = _bcast(par_v, 6)
    e0 = _bcast(par_v, 7); e1 = _bcast(par_v, 8); e2 = _bcast(par_v, 9)
    col0 = jnp.full((16,), 0, jnp.int32)
    col1 = jnp.full((16,), 1, jnp.int32)
    col2 = jnp.full((16,), 2, jnp.int32)
    col3 = jnp.full((16,), 3, jnp.int32)
    zf = jnp.zeros((16,), jnp.float32)

    def grp(g, carry):
        row = g * 16 + lane
        xs = plsc.load_gather(a0, [row, col0]) + plsc.load_gather(a1, [row, col0])
        ys = plsc.load_gather(a0, [row, col1]) + plsc.load_gather(a1, [row, col1])
        zs = plsc.load_gather(a0, [row, col2]) + plsc.load_gather(a1, [row, col2])
        cn = plsc.load_gather(a0, [row, col3]) + plsc.load_gather(a1, [row, col3])
        cnt = jnp.maximum(cn, 1.0)
        fx = xs / cnt + b0
        fy = ys / cnt + b1
        fz = zs / cnt + b2
        mu = (fx + fy + fz) * jnp.float32(1.0 / 3.0)
        ex = fx - mu
        ey = fy - mu
        ez = fz - mu
        var = (ex * ex + ey * ey + ez * ez) * jnp.float32(1.0 / 3.0)
        rs = _rsqrt(var + 1e-5)
        plsc.store_scatter(fo, [row, col0], ex * rs * g0 + e0)
        plsc.store_scatter(fo, [row, col1], ey * rs * g1 + e1)
        plsc.store_scatter(fo, [row, col2], ez * rs * g2 + e2)
        plsc.store_scatter(fo, [row, col3], zf)
        return carry

    lax.fori_loop(0, NODES_W // 16, grp, 0)
    pltpu.sync_copy(fo, feat_hbm.at[pl.ds(nb, NODES_W), :])


@functools.partial(
    pl.kernel,
    compiler_params=_CP,
    out_type=[jax.ShapeDtypeStruct((N_EDGES,), jnp.float32)] * 3,
    mesh=_MESH,
    scratch_types=[
        pltpu.VMEM((NCH, CH), jnp.int32),
        pltpu.VMEM((BLK, RW), jnp.float32),
        pltpu.VMEM((BLK,), jnp.float32),
        pltpu.VMEM((BLK,), jnp.float32),
        pltpu.VMEM((BLK,), jnp.float32),
        pltpu.SemaphoreType.DMA,
        pltpu.SemaphoreType.DMA,
    ],
)
def _gather_pass(feat_hbm, idx_hbm, ox_hbm, oy_hbm, oz_hbm,
                 idx_v, rows_v, ox_v, oy_v, oz_v, sem, sem_out):
    c = lax.axis_index("c")
    s = lax.axis_index("s")
    w = c * NS + s
    lane = lax.iota(jnp.int32, 16)
    col0 = jnp.full((16,), 0, jnp.int32)
    col1 = jnp.full((16,), 1, jnp.int32)
    col2 = jnp.full((16,), 2, jnp.int32)

    def block_body(k, carry):
        blk = k * NW + w

        @pl.when(blk < NBLKS)
        def _():
            eb = pl.multiple_of(blk * BLK, 8)
            pltpu.sync_copy(
                idx_hbm.at[pl.ds(pl.multiple_of(blk * NCH, 8), NCH), :], idx_v)
            copies = [
                pltpu.async_copy(feat_hbm.at[idx_v.at[j]],
                                 rows_v.at[pl.ds(CH * j, CH), :], sem)
                for j in range(NCH)
            ]
            for d in copies:
                d.wait()

            def grp(g, carry2):
                o = g * 16
                row = o + lane
                ox_v[pl.ds(o, 16)] = plsc.load_gather(rows_v, [row, col0])
                oy_v[pl.ds(o, 16)] = plsc.load_gather(rows_v, [row, col1])
                oz_v[pl.ds(o, 16)] = plsc.load_gather(rows_v, [row, col2])
                return carry2

            lax.fori_loop(0, BLK // 16, grp, 0)

            stores = [
                pltpu.async_copy(ox_v, ox_hbm.at[pl.ds(eb, BLK)], sem_out),
                pltpu.async_copy(oy_v, oy_hbm.at[pl.ds(eb, BLK)], sem_out),
                pltpu.async_copy(oz_v, oz_hbm.at[pl.ds(eb, BLK)], sem_out),
            ]
            for d in stores:
                d.wait()

        return carry

    lax.fori_loop(0, KMAX, block_body, 0)


def kernel(relative_pos, edge_index_i, kernel_dirs, W, b, ln_gamma, ln_beta):
    m3 = kernel_dirs.T @ W.T
    pad1 = jnp.zeros((1,), jnp.float32)
    m3_pad = jnp.concatenate([pad1, m3.reshape(9), jnp.zeros((6,), jnp.float32)])
    params = jnp.concatenate(
        [pad1, b, ln_gamma, ln_beta, jnp.zeros((6,), jnp.float32)])

    px = relative_pos[:, 0]
    py = relative_pos[:, 1]
    pz = relative_pos[:, 2]
    idx2d = edge_index_i.reshape(NROW, CH)
    zeros = jnp.zeros((N_PAD, RW), jnp.float32)

    acc = _scatter_pass(px, py, pz, idx2d, zeros, m3_pad)
    feat = _node_pass(acc, params)
    ox, oy, oz = _gather_pass(feat, idx2d)
    return jnp.stack([ox, oy, oz], axis=1)


# speedup vs baseline: 40.1608x; 1.0756x over previous
import functools

import jax
import jax.numpy as jnp
from jax import lax
from jax.experimental import pallas as pl
from jax.experimental.pallas import tpu as pltpu
from jax.experimental.pallas import tpu_sc as plsc

N_NODES = 100000
N_EDGES = 6400000
NC = 2
NS = 16
NW = NC * NS
CH = 128
NCH = 32
BLK = NCH * CH
NBLKS = N_EDGES // BLK
KMAX = -(-NBLKS // NW)
NROW = N_EDGES // CH
N_PAD = NW * 3136
NODES_W = N_PAD // NW
NODES_S = N_PAD // NS
RW = 8


def _rsqrt(x):
    i = lax.bitcast_convert_type(x, jnp.int32)
    i = jnp.int32(0x5F3759DF) - lax.shift_right_arithmetic(i, 1)
    y = lax.bitcast_convert_type(i, jnp.float32)
    y = y * (1.5 - 0.5 * x * y * y)
    y = y * (1.5 - 0.5 * x * y * y)
    return y


def _bcast(vec_ref, k):
    assert k >= 1
    return plsc.load_gather(vec_ref, [jnp.full((16,), k, jnp.int32)])


_MESH = plsc.VectorSubcoreMesh(core_axis_name="c", subcore_axis_name="s",
                               num_cores=NC, num_subcores=NS)
_CP = pltpu.CompilerParams(needs_layout_passes=False, use_tc_tiling_on_sc=False)


@functools.partial(
    pl.kernel,
    compiler_params=_CP,
    out_type=jax.ShapeDtypeStruct((NC, N_PAD, RW), jnp.float32),
    mesh=_MESH,
    scratch_types=[
        pltpu.VMEM((BLK,), jnp.float32),
        pltpu.VMEM((BLK,), jnp.float32),
        pltpu.VMEM((BLK,), jnp.float32),
        pltpu.VMEM((NCH, CH), jnp.int32),
        pltpu.VMEM((BLK, RW), jnp.float32),
        pltpu.VMEM((16,), jnp.float32),
        pltpu.VMEM_SHARED((N_PAD, RW), jnp.float32),
        pltpu.SemaphoreType.DMA,
        pltpu.SemaphoreType.DMA,
    ],
)
def _scatter_pass(px_hbm, py_hbm, pz_hbm, idx_hbm, zeros_hbm, m3_hbm,
                  acc_hbm, px_v, py_v, pz_v, idx_v, contrib, m_v, acc_sh, sem,
                  sem_in):
    c = lax.axis_index("c")
    s = lax.axis_index("s")
    w = c * NS + s

    pltpu.sync_copy(zeros_hbm.at[pl.ds(NODES_S * s, NODES_S), :],
                    acc_sh.at[pl.ds(NODES_S * s, NODES_S), :])
    pltpu.sync_copy(m3_hbm, m_v)
    pltpu.sync_copy(zeros_hbm.at[pl.ds(0, BLK), :], contrib)
    plsc.subcore_barrier()

    lane = lax.iota(jnp.int32, 16)
    m00 = _bcast(m_v, 1); m01 = _bcast(m_v, 2); m02 = _bcast(m_v, 3)
    m10 = _bcast(m_v, 4); m11 = _bcast(m_v, 5); m12 = _bcast(m_v, 6)
    m20 = _bcast(m_v, 7); m21 = _bcast(m_v, 8); m22 = _bcast(m_v, 9)
    col0 = jnp.full((16,), 0, jnp.int32)
    col1 = jnp.full((16,), 1, jnp.int32)
    col2 = jnp.full((16,), 2, jnp.int32)
    col3 = jnp.full((16,), 3, jnp.int32)
    ones = jnp.full((16,), 1.0, jnp.float32)

    def block_body(k, carry):
        blk = k * NW + w

        @pl.when(blk < NBLKS)
        def _():
            eb = pl.multiple_of(blk * BLK, 8)
            loads = [
                pltpu.async_copy(px_hbm.at[pl.ds(eb, BLK)], px_v, sem_in),
                pltpu.async_copy(py_hbm.at[pl.ds(eb, BLK)], py_v, sem_in),
                pltpu.async_copy(pz_hbm.at[pl.ds(eb, BLK)], pz_v, sem_in),
                pltpu.async_copy(
                    idx_hbm.at[pl.ds(pl.multiple_of(blk * NCH, 8), NCH), :],
                    idx_v, sem_in),
            ]
            for d in loads:
                d.wait()

            def grp(g, carry2):
                o = g * 16
                px = px_v[pl.ds(o, 16)]
                py = py_v[pl.ds(o, 16)]
                pz = pz_v[pl.ds(o, 16)]
                n2 = px * px + py * py + pz * pz
                r = _rsqrt(n2)
                inv = 1.0 / (n2 * r + 1e-8)
                dx = px * inv
                dy = py * inv
                dz = pz * inv
                cx = dx * m00 + dy * m10 + dz * m20
                cy = dx * m01 + dy * m11 + dz * m21
                cz = dx * m02 + dy * m12 + dz * m22
                p16 = o + lane
                plsc.store_scatter(contrib, [p16, col0], cx)
                plsc.store_scatter(contrib, [p16, col1], cy)
                plsc.store_scatter(contrib, [p16, col2], cz)
                plsc.store_scatter(contrib, [p16, col3], ones)
                return carry2

            lax.fori_loop(0, BLK // 16, grp, 0)

            copies = [
                pltpu.async_copy(contrib.at[pl.ds(CH * j, CH), :],
                                 acc_sh.at[idx_v.at[j]], sem, add=True)
                for j in range(NCH)
            ]
            for d in copies:
                d.wait()

        return carry

    lax.fori_loop(0, KMAX, block_body, 0)
    plsc.subcore_barrier()
    pltpu.sync_copy(acc_sh.at[pl.ds(NODES_S * s, NODES_S), :],
                    acc_hbm.at[c, pl.ds(NODES_S * s, NODES_S), :])


@functools.partial(
    pl.kernel,
    compiler_params=_CP,
    out_type=jax.ShapeDtypeStruct((N_PAD, RW), jnp.float32),
    mesh=_MESH,
    scratch_types=[
        pltpu.VMEM((NODES_W, RW), jnp.float32),
        pltpu.VMEM((NODES_W, RW), jnp.float32),
        pltpu.VMEM((NODES_W, RW), jnp.float32),
        pltpu.VMEM((16,), jnp.float32),
    ],
)
def _node_pass(acc_hbm, par_hbm, feat_hbm, a0, a1, fo, par_v):
    c = lax.axis_index("c")
    s = lax.axis_index("s")
    w = c * NS + s
    nb = w * NODES_W
    pltpu.sync_copy(acc_hbm.at[0, pl.ds(nb, NODES_W), :], a0)
    pltpu.sync_copy(acc_hbm.at[1, pl.ds(nb, NODES_W), :], a1)
    pltpu.sync_copy(par_hbm, par_v)

    lane = lax.iota(jnp.int32, 16)
    b0 = _bcast(par_v, 1); b1 = _bcast(par_v, 2); b2 = _bcast(par_v, 3)
    g0 = _bcast(par_v, 4); g1 = _bcast(par_v, 5); g2 = _bcast(par_v, 6)
    e0 = _bcast(par_v, 7); e1 = _bcast(par_v, 8); e2 = _bcast(par_v, 9)
    col0 = jnp.full((16,), 0, jnp.int32)
    col1 = jnp.full((16,), 1, jnp.int32)
    col2 = jnp.full((16,), 2, jnp.int32)
    col3 = jnp.full((16,), 3, jnp.int32)
    zf = jnp.zeros((16,), jnp.float32)

    def grp(g, carry):
        row = g * 16 + lane
        xs = plsc.load_gather(a0, [row, col0]) + plsc.load_gather(a1, [row, col0])
        ys = plsc.load_gather(a0, [row, col1]) + plsc.load_gather(a1, [row, col1])
        zs = plsc.load_gather(a0, [row, col2]) + plsc.load_gather(a1, [row, col2])
        cn = plsc.load_gather(a0, [row, col3]) + plsc.load_gather(a1, [row, col3])
        cnt = jnp.maximum(cn, 1.0)
        fx = xs / cnt + b0
        fy = ys / cnt + b1
        fz = zs / cnt + b2
        mu = (fx + fy + fz) * jnp.float32(1.0 / 3.0)
        ex = fx - mu
        ey = fy - mu
        ez = fz - mu
        var = (ex * ex + ey * ey + ez * ez) * jnp.float32(1.0 / 3.0)
        rs = _rsqrt(var + 1e-5)
        plsc.store_scatter(fo, [row, col0], ex * rs * g0 + e0)
        plsc.store_scatter(fo, [row, col1], ey * rs * g1 + e1)
        plsc.store_scatter(fo, [row, col2], ez * rs * g2 + e2)
        plsc.store_scatter(fo, [row, col3], zf)
        return carry

    lax.fori_loop(0, NODES_W // 16, grp, 0)
    pltpu.sync_copy(fo, feat_hbm.at[pl.ds(nb, NODES_W), :])


@functools.partial(
    pl.kernel,
    compiler_params=_CP,
    out_type=[jax.ShapeDtypeStruct((N_EDGES,), jnp.float32)] * 3,
    mesh=_MESH,
    scratch_types=[
        pltpu.VMEM((NCH, CH), jnp.int32),
        pltpu.VMEM((BLK, RW), jnp.float32),
        pltpu.VMEM((BLK,), jnp.float32),
        pltpu.VMEM((BLK,), jnp.float32),
        pltpu.VMEM((BLK,), jnp.float32),
        pltpu.SemaphoreType.DMA,
        pltpu.SemaphoreType.DMA,
    ],
)
def _gather_pass(feat_hbm, idx_hbm, ox_hbm, oy_hbm, oz_hbm,
                 idx_v, rows_v, ox_v, oy_v, oz_v, sem, sem_out):
    c = lax.axis_index("c")
    s = lax.axis_index("s")
    w = c * NS + s
    lane = lax.iota(jnp.int32, 16)
    col0 = jnp.full((16,), 0, jnp.int32)
    col1 = jnp.full((16,), 1, jnp.int32)
    col2 = jnp.full((16,), 2, jnp.int32)

    def block_body(k, carry):
        blk = k * NW + w

        @pl.when(blk < NBLKS)
        def _():
            eb = pl.multiple_of(blk * BLK, 8)
            pltpu.sync_copy(
                idx_hbm.at[pl.ds(pl.multiple_of(blk * NCH, 8), NCH), :], idx_v)
            copies = [
                pltpu.async_copy(feat_hbm.at[idx_v.at[j]],
                                 rows_v.at[pl.ds(CH * j, CH), :], sem)
                for j in range(NCH)
            ]
            for d in copies:
                d.wait()

            def grp(g, carry2):
                o = g * 16
                row = o + lane
                ox_v[pl.ds(o, 16)] = plsc.load_gather(rows_v, [row, col0])
                oy_v[pl.ds(o, 16)] = plsc.load_gather(rows_v, [row, col1])
                oz_v[pl.ds(o, 16)] = plsc.load_gather(rows_v, [row, col2])
                return carry2

            lax.fori_loop(0, BLK // 16, grp, 0)

            stores = [
                pltpu.async_copy(ox_v, ox_hbm.at[pl.ds(eb, BLK)], sem_out),
                pltpu.async_copy(oy_v, oy_hbm.at[pl.ds(eb, BLK)], sem_out),
                pltpu.async_copy(oz_v, oz_hbm.at[pl.ds(eb, BLK)], sem_out),
            ]
            for d in stores:
                d.wait()

        return carry

    lax.fori_loop(0, KMAX, block_body, 0)


def kernel(relative_pos, edge_index_i, kernel_dirs, W, b, ln_gamma, ln_beta):
    m3 = kernel_dirs.T @ W.T
    pad1 = jnp.zeros((1,), jnp.float32)
    m3_pad = jnp.concatenate([pad1, m3.reshape(9), jnp.zeros((6,), jnp.float32)])
    params = jnp.concatenate(
        [pad1, b, ln_gamma, ln_beta, jnp.zeros((6,), jnp.float32)])

    px = relative_pos[:, 0]
    py = relative_pos[:, 1]
    pz = relative_pos[:, 2]
    idx2d = edge_index_i.reshape(NROW, CH)
    zeros = jnp.zeros((N_PAD, RW), jnp.float32)

    acc = _scatter_pass(px, py, pz, idx2d, zeros, m3_pad)
    feat = _node_pass(acc, params)
    ox, oy, oz = _gather_pass(feat, idx2d)
    return jnp.stack([ox, oy, oz], axis=1)


# speedup vs baseline: 41.5901x; 1.0356x over previous
import functools

import jax
import jax.numpy as jnp
from jax import lax
from jax.experimental import pallas as pl
from jax.experimental.pallas import tpu as pltpu
from jax.experimental.pallas import tpu_sc as plsc

N_NODES = 100000
N_EDGES = 6400000
NC = 2
NS = 16
NW = NC * NS
CH = 128
NCH = 16
BLK = NCH * CH
NBLKS = N_EDGES // BLK
KMAX = -(-NBLKS // NW)
NROW = N_EDGES // CH
N_PAD = NW * 3136
NODES_W = N_PAD // NW
NODES_S = N_PAD // NS
RW = 8


def _rsqrt(x):
    i = lax.bitcast_convert_type(x, jnp.int32)
    i = jnp.int32(0x5F3759DF) - lax.shift_right_arithmetic(i, 1)
    y = lax.bitcast_convert_type(i, jnp.float32)
    y = y * (1.5 - 0.5 * x * y * y)
    y = y * (1.5 - 0.5 * x * y * y)
    return y


def _bcast(vec_ref, k):
    assert k >= 1
    return plsc.load_gather(vec_ref, [jnp.full((16,), k, jnp.int32)])


_MESH = plsc.VectorSubcoreMesh(core_axis_name="c", subcore_axis_name="s",
                               num_cores=NC, num_subcores=NS)
_CP = pltpu.CompilerParams(needs_layout_passes=False, use_tc_tiling_on_sc=False)


@functools.partial(
    pl.kernel,
    compiler_params=_CP,
    out_type=jax.ShapeDtypeStruct((NC, N_PAD, RW), jnp.float32),
    mesh=_MESH,
    scratch_types=[
        pltpu.VMEM((2, BLK), jnp.float32),
        pltpu.VMEM((2, BLK), jnp.float32),
        pltpu.VMEM((2, BLK), jnp.float32),
        pltpu.VMEM((3, NCH, CH), jnp.int32),
        pltpu.VMEM((2, BLK, RW), jnp.float32),
        pltpu.VMEM((16,), jnp.float32),
        pltpu.VMEM_SHARED((N_PAD, RW), jnp.float32),
        pltpu.SemaphoreType.DMA,
        pltpu.SemaphoreType.DMA,
        pltpu.SemaphoreType.DMA,
    ],
)
def _scatter_pass(px_hbm, py_hbm, pz_hbm, idx_hbm, zeros_hbm, m3_hbm,
                  acc_hbm, px_v, py_v, pz_v, idx_v, contrib, m_v, acc_sh,
                  sem_in0, sem_in1, sem_st):
    c = lax.axis_index("c")
    s = lax.axis_index("s")
    w = c * NS + s
    sem_in = (sem_in0, sem_in1)

    pltpu.sync_copy(zeros_hbm.at[pl.ds(NODES_S * s, NODES_S), :],
                    acc_sh.at[pl.ds(NODES_S * s, NODES_S), :])
    pltpu.sync_copy(m3_hbm, m_v)
    pltpu.sync_copy(zeros_hbm.at[pl.ds(0, BLK), :], contrib.at[0])
    pltpu.sync_copy(zeros_hbm.at[pl.ds(0, BLK), :], contrib.at[1])
    plsc.subcore_barrier()

    lane = lax.iota(jnp.int32, 16)
    m00 = _bcast(m_v, 1); m01 = _bcast(m_v, 2); m02 = _bcast(m_v, 3)
    m10 = _bcast(m_v, 4); m11 = _bcast(m_v, 5); m12 = _bcast(m_v, 6)
    m20 = _bcast(m_v, 7); m21 = _bcast(m_v, 8); m22 = _bcast(m_v, 9)
    col0 = jnp.full((16,), 0, jnp.int32)
    col1 = jnp.full((16,), 1, jnp.int32)
    col2 = jnp.full((16,), 2, jnp.int32)
    col3 = jnp.full((16,), 3, jnp.int32)
    ones = jnp.full((16,), 1.0, jnp.float32)

    def in_descs(blk, par, tri, sem):
        eb = pl.multiple_of(blk * BLK, 8)
        rb = pl.multiple_of(blk * NCH, 8)
        return [
            pltpu.make_async_copy(px_hbm.at[pl.ds(eb, BLK)], px_v.at[par], sem),
            pltpu.make_async_copy(py_hbm.at[pl.ds(eb, BLK)], py_v.at[par], sem),
            pltpu.make_async_copy(pz_hbm.at[pl.ds(eb, BLK)], pz_v.at[par], sem),
            pltpu.make_async_copy(idx_hbm.at[pl.ds(rb, NCH), :],
                                  idx_v.at[tri], sem),
        ]

    def st_descs(par, tri):
        return [
            pltpu.make_async_copy(contrib.at[par, pl.ds(CH * j, CH), :],
                                  acc_sh.at[idx_v.at[tri, j]], sem_st)
            for j in range(NCH)
        ]

    def compute(par):
        def grp(g, carry2):
            o = g * 16
            px = px_v[par, pl.ds(o, 16)]
            py = py_v[par, pl.ds(o, 16)]
            pz = pz_v[par, pl.ds(o, 16)]
            n2 = px * px + py * py + pz * pz
            r = _rsqrt(n2)
            inv = 1.0 / (n2 * r + 1e-8)
            dx = px * inv
            dy = py * inv
            dz = pz * inv
            cx = dx * m00 + dy * m10 + dz * m20
            cy = dx * m01 + dy * m11 + dz * m21
            cz = dx * m02 + dy * m12 + dz * m22
            p16 = o + lane
            cb = contrib.at[par]
            plsc.store_scatter(cb, [p16, col0], cx)
            plsc.store_scatter(cb, [p16, col1], cy)
            plsc.store_scatter(cb, [p16, col2], cz)
            plsc.store_scatter(cb, [p16, col3], ones)
            return carry2

        lax.fori_loop(0, BLK // 16, grp, 0)

    for d in in_descs(w, 0, 0, sem_in[0]):
        d.start()

    def sup(t, carry):
        for u in range(6):
            i6 = t * 6 + u
            par, tri = u % 2, u % 3
            blk = i6 * NW + w
            nblk = blk + NW

            @pl.when(nblk < NBLKS)
            def _():
                for d in in_descs(nblk, (u + 1) % 2, (u + 1) % 3,
                                  sem_in[(u + 1) % 2]):
                    d.start()

            @pl.when(blk < NBLKS)
            def _():
                for d in in_descs(blk, par, tri, sem_in[par]):
                    d.wait()
                compute(par)

            @pl.when((blk - NW < NBLKS) & (i6 >= 1))
            def _():
                for d in st_descs((u - 1) % 2, (u - 1) % 3):
                    d.wait()

            @pl.when(blk < NBLKS)
            def _():
                for d in st_descs(par, tri):
                    d.start(add=True)

        return carry

    lax.fori_loop(0, (KMAX + 1 + 5) // 6, sup, 0)
    plsc.subcore_barrier()
    pltpu.sync_copy(acc_sh.at[pl.ds(NODES_S * s, NODES_S), :],
                    acc_hbm.at[c, pl.ds(NODES_S * s, NODES_S), :])


@functools.partial(
    pl.kernel,
    compiler_params=_CP,
    out_type=jax.ShapeDtypeStruct((N_PAD, RW), jnp.float32),
    mesh=_MESH,
    scratch_types=[
        pltpu.VMEM((NODES_W, RW), jnp.float32),
        pltpu.VMEM((NODES_W, RW), jnp.float32),
        pltpu.VMEM((NODES_W, RW), jnp.float32),
        pltpu.VMEM((16,), jnp.float32),
    ],
)
def _node_pass(acc_hbm, par_hbm, feat_hbm, a0, a1, fo, par_v):
    c = lax.axis_index("c")
    s = lax.axis_index("s")
    w = c * NS + s
    nb = w * NODES_W
    pltpu.sync_copy(acc_hbm.at[0, pl.ds(nb, NODES_W), :], a0)
    pltpu.sync_copy(acc_hbm.at[1, pl.ds(nb, NODES_W), :], a1)
    pltpu.sync_copy(par_hbm, par_v)

    lane = lax.iota(jnp.int32, 16)
    b0 = _bcast(par_v, 1); b1 = _bcast(par_v, 2); b2 = _bcast(par_v, 3)
    g0 = _bcast(par_v, 4); g1 = _bcast(par_v, 5); g2 = _bcast(par_v, 6)
    e0 = _bcast(par_v, 7); e1 = _bcast(par_v, 8); e2 = _bcast(par_v, 9)
    col0 = jnp.full((16,), 0, jnp.int32)
    col1 = jnp.full((16,), 1, jnp.int32)
    col2 = jnp.full((16,), 2, jnp.int32)
    col3 = jnp.full((16,), 3, jnp.int32)
    zf = jnp.zeros((16,), jnp.float32)

    def grp(g, carry):
        row = g * 16 + lane
        xs = plsc.load_gather(a0, [row, col0]) + plsc.load_gather(a1, [row, col0])
        ys = plsc.load_gather(a0, [row, col1]) + plsc.load_gather(a1, [row, col1])
        zs = plsc.load_gather(a0, [row, col2]) + plsc.load_gather(a1, [row, col2])
        cn = plsc.load_gather(a0, [row, col3]) + plsc.load_gather(a1, [row, col3])
        cnt = jnp.maximum(cn, 1.0)
        fx = xs / cnt + b0
        fy = ys / cnt + b1
        fz = zs / cnt + b2
        mu = (fx + fy + fz) * jnp.float32(1.0 / 3.0)
        ex = fx - mu
        ey = fy - mu
        ez = fz - mu
        var = (ex * ex + ey * ey + ez * ez) * jnp.float32(1.0 / 3.0)
        rs = _rsqrt(var + 1e-5)
        plsc.store_scatter(fo, [row, col0], ex * rs * g0 + e0)
        plsc.store_scatter(fo, [row, col1], ey * rs * g1 + e1)
        plsc.store_scatter(fo, [row, col2], ez * rs * g2 + e2)
        plsc.store_scatter(fo, [row, col3], zf)
        return carry

    lax.fori_loop(0, NODES_W // 16, grp, 0)
    pltpu.sync_copy(fo, feat_hbm.at[pl.ds(nb, NODES_W), :])


@functools.partial(
    pl.kernel,
    compiler_params=_CP,
    out_type=[jax.ShapeDtypeStruct((N_EDGES,), jnp.float32)] * 3,
    mesh=_MESH,
    scratch_types=[
        pltpu.VMEM((NCH, CH), jnp.int32),
        pltpu.VMEM((BLK, RW), jnp.float32),
        pltpu.VMEM((BLK,), jnp.float32),
        pltpu.VMEM((BLK,), jnp.float32),
        pltpu.VMEM((BLK,), jnp.float32),
        pltpu.SemaphoreType.DMA,
        pltpu.SemaphoreType.DMA,
    ],
)
def _gather_pass(feat_hbm, idx_hbm, ox_hbm, oy_hbm, oz_hbm,
                 idx_v, rows_v, ox_v, oy_v, oz_v, sem, sem_out):
    c = lax.axis_index("c")
    s = lax.axis_index("s")
    w = c * NS + s
    lane = lax.iota(jnp.int32, 16)
    col0 = jnp.full((16,), 0, jnp.int32)
    col1 = jnp.full((16,), 1, jnp.int32)
    col2 = jnp.full((16,), 2, jnp.int32)

    def block_body(k, carry):
        blk = k * NW + w

        @pl.when(blk < NBLKS)
        def _():
            eb = pl.multiple_of(blk * BLK, 8)
            pltpu.sync_copy(
                idx_hbm.at[pl.ds(pl.multiple_of(blk * NCH, 8), NCH), :], idx_v)
            copies = [
                pltpu.async_copy(feat_hbm.at[idx_v.at[j]],
                                 rows_v.at[pl.ds(CH * j, CH), :], sem)
                for j in range(NCH)
            ]
            for d in copies:
                d.wait()

            def grp(g, carry2):
                o = g * 16
                row = o + lane
                ox_v[pl.ds(o, 16)] = plsc.load_gather(rows_v, [row, col0])
                oy_v[pl.ds(o, 16)] = plsc.load_gather(rows_v, [row, col1])
                oz_v[pl.ds(o, 16)] = plsc.load_gather(rows_v, [row, col2])
                return carry2

            lax.fori_loop(0, BLK // 16, grp, 0)

            stores = [
                pltpu.async_copy(ox_v, ox_hbm.at[pl.ds(eb, BLK)], sem_out),
                pltpu.async_copy(oy_v, oy_hbm.at[pl.ds(eb, BLK)], sem_out),
                pltpu.async_copy(oz_v, oz_hbm.at[pl.ds(eb, BLK)], sem_out),
            ]
            for d in stores:
                d.wait()

        return carry

    lax.fori_loop(0, KMAX, block_body, 0)


def kernel(relative_pos, edge_index_i, kernel_dirs, W, b, ln_gamma, ln_beta):
    m3 = kernel_dirs.T @ W.T
    pad1 = jnp.zeros((1,), jnp.float32)
    m3_pad = jnp.concatenate([pad1, m3.reshape(9), jnp.zeros((6,), jnp.float32)])
    params = jnp.concatenate(
        [pad1, b, ln_gamma, ln_beta, jnp.zeros((6,), jnp.float32)])

    px = relative_pos[:, 0]
    py = relative_pos[:, 1]
    pz = relative_pos[:, 2]
    idx2d = edge_index_i.reshape(NROW, CH)
    zeros = jnp.zeros((N_PAD, RW), jnp.float32)

    acc = _scatter_pass(px, py, pz, idx2d, zeros, m3_pad)
    feat = _node_pass(acc, params)
    ox, oy, oz = _gather_pass(feat, idx2d)
    return jnp.stack([ox, oy, oz], axis=1)


# speedup vs baseline: 55.9032x; 1.3441x over previous
import functools

import jax
import jax.numpy as jnp
from jax import lax
from jax.experimental import pallas as pl
from jax.experimental.pallas import tpu as pltpu
from jax.experimental.pallas import tpu_sc as plsc

N_NODES = 100000
N_EDGES = 6400000
NC = 2
NS = 16
NW = NC * NS
CH = 128
NCH = 16
BLK = NCH * CH
NBLKS = N_EDGES // BLK
KMAX = -(-NBLKS // NW)
NROW = N_EDGES // CH
N_PAD = NW * 3136
NODES_W = N_PAD // NW
NODES_S = N_PAD // NS
RW = 8


def _rsqrt(x):
    i = lax.bitcast_convert_type(x, jnp.int32)
    i = jnp.int32(0x5F3759DF) - lax.shift_right_arithmetic(i, 1)
    y = lax.bitcast_convert_type(i, jnp.float32)
    y = y * (1.5 - 0.5 * x * y * y)
    y = y * (1.5 - 0.5 * x * y * y)
    return y


def _bcast(vec_ref, k):
    assert k >= 1
    return plsc.load_gather(vec_ref, [jnp.full((16,), k, jnp.int32)])


_MESH = plsc.VectorSubcoreMesh(core_axis_name="c", subcore_axis_name="s",
                               num_cores=NC, num_subcores=NS)
_CP = pltpu.CompilerParams(needs_layout_passes=False, use_tc_tiling_on_sc=False)


@functools.partial(
    pl.kernel,
    compiler_params=_CP,
    out_type=jax.ShapeDtypeStruct((NC, N_PAD, RW), jnp.float32),
    mesh=_MESH,
    scratch_types=[
        pltpu.VMEM((2, BLK), jnp.float32),
        pltpu.VMEM((2, BLK), jnp.float32),
        pltpu.VMEM((2, BLK), jnp.float32),
        pltpu.VMEM((3, NCH, CH), jnp.int32),
        pltpu.VMEM((2, BLK, RW), jnp.float32),
        pltpu.VMEM((16,), jnp.float32),
        pltpu.VMEM_SHARED((N_PAD, RW), jnp.float32),
        pltpu.SemaphoreType.DMA,
        pltpu.SemaphoreType.DMA,
        pltpu.SemaphoreType.DMA,
    ],
)
def _scatter_pass(px_hbm, py_hbm, pz_hbm, idx_hbm, zeros_hbm, m3_hbm,
                  acc_hbm, px_v, py_v, pz_v, idx_v, contrib, m_v, acc_sh,
                  sem_in0, sem_in1, sem_st):
    c = lax.axis_index("c")
    s = lax.axis_index("s")
    w = c * NS + s
    sem_in = (sem_in0, sem_in1)

    pltpu.sync_copy(zeros_hbm.at[pl.ds(NODES_S * s, NODES_S), :],
                    acc_sh.at[pl.ds(NODES_S * s, NODES_S), :])
    pltpu.sync_copy(m3_hbm, m_v)
    pltpu.sync_copy(zeros_hbm.at[pl.ds(0, BLK), :], contrib.at[0])
    pltpu.sync_copy(zeros_hbm.at[pl.ds(0, BLK), :], contrib.at[1])
    plsc.subcore_barrier()

    lane = lax.iota(jnp.int32, 16)
    m00 = _bcast(m_v, 1); m01 = _bcast(m_v, 2); m02 = _bcast(m_v, 3)
    m10 = _bcast(m_v, 4); m11 = _bcast(m_v, 5); m12 = _bcast(m_v, 6)
    m20 = _bcast(m_v, 7); m21 = _bcast(m_v, 8); m22 = _bcast(m_v, 9)
    col0 = jnp.full((16,), 0, jnp.int32)
    col1 = jnp.full((16,), 1, jnp.int32)
    col2 = jnp.full((16,), 2, jnp.int32)
    col3 = jnp.full((16,), 3, jnp.int32)
    ones = jnp.full((16,), 1.0, jnp.float32)

    def in_descs(blk, par, tri, sem):
        eb = pl.multiple_of(blk * BLK, 8)
        rb = pl.multiple_of(blk * NCH, 8)
        return [
            pltpu.make_async_copy(px_hbm.at[pl.ds(eb, BLK)], px_v.at[par], sem),
            pltpu.make_async_copy(py_hbm.at[pl.ds(eb, BLK)], py_v.at[par], sem),
            pltpu.make_async_copy(pz_hbm.at[pl.ds(eb, BLK)], pz_v.at[par], sem),
            pltpu.make_async_copy(idx_hbm.at[pl.ds(rb, NCH), :],
                                  idx_v.at[tri], sem),
        ]

    def st_descs(par, tri):
        return [
            pltpu.make_async_copy(contrib.at[par, pl.ds(CH * j, CH), :],
                                  acc_sh.at[idx_v.at[tri, j]], sem_st)
            for j in range(NCH)
        ]

    def compute(par):
        def grp(g, carry2):
            o = g * 16
            px = px_v[par, pl.ds(o, 16)]
            py = py_v[par, pl.ds(o, 16)]
            pz = pz_v[par, pl.ds(o, 16)]
            n2 = px * px + py * py + pz * pz
            r = _rsqrt(n2)
            inv = 1.0 / (n2 * r + 1e-8)
            dx = px * inv
            dy = py * inv
            dz = pz * inv
            cx = dx * m00 + dy * m10 + dz * m20
            cy = dx * m01 + dy * m11 + dz * m21
            cz = dx * m02 + dy * m12 + dz * m22
            p16 = o + lane
            cb = contrib.at[par]
            plsc.store_scatter(cb, [p16, col0], cx)
            plsc.store_scatter(cb, [p16, col1], cy)
            plsc.store_scatter(cb, [p16, col2], cz)
            plsc.store_scatter(cb, [p16, col3], ones)
            return carry2

        lax.fori_loop(0, BLK // 16, grp, 0)

    for d in in_descs(w, 0, 0, sem_in[0]):
        d.start()

    def sup(t, carry):
        for u in range(6):
            i6 = t * 6 + u
            par, tri = u % 2, u % 3
            blk = i6 * NW + w
            nblk = blk + NW

            @pl.when(nblk < NBLKS)
            def _():
                for d in in_descs(nblk, (u + 1) % 2, (u + 1) % 3,
                                  sem_in[(u + 1) % 2]):
                    d.start()

            @pl.when(blk < NBLKS)
            def _():
                for d in in_descs(blk, par, tri, sem_in[par]):
                    d.wait()
                compute(par)

            @pl.when((blk - NW < NBLKS) & (i6 >= 1))
            def _():
                for d in st_descs((u - 1) % 2, (u - 1) % 3):
                    d.wait()

            @pl.when(blk < NBLKS)
            def _():
                for d in st_descs(par, tri):
                    d.start(add=True)

        return carry

    lax.fori_loop(0, (KMAX + 1 + 5) // 6, sup, 0)
    plsc.subcore_barrier()
    pltpu.sync_copy(acc_sh.at[pl.ds(NODES_S * s, NODES_S), :],
                    acc_hbm.at[c, pl.ds(NODES_S * s, NODES_S), :])


@functools.partial(
    pl.kernel,
    compiler_params=_CP,
    out_type=jax.ShapeDtypeStruct((N_PAD, RW), jnp.float32),
    mesh=_MESH,
    scratch_types=[
        pltpu.VMEM((NODES_W, RW), jnp.float32),
        pltpu.VMEM((NODES_W, RW), jnp.float32),
        pltpu.VMEM((NODES_W, RW), jnp.float32),
        pltpu.VMEM((16,), jnp.float32),
    ],
)
def _node_pass(acc_hbm, par_hbm, feat_hbm, a0, a1, fo, par_v):
    c = lax.axis_index("c")
    s = lax.axis_index("s")
    w = c * NS + s
    nb = w * NODES_W
    pltpu.sync_copy(acc_hbm.at[0, pl.ds(nb, NODES_W), :], a0)
    pltpu.sync_copy(acc_hbm.at[1, pl.ds(nb, NODES_W), :], a1)
    pltpu.sync_copy(par_hbm, par_v)

    lane = lax.iota(jnp.int32, 16)
    b0 = _bcast(par_v, 1); b1 = _bcast(par_v, 2); b2 = _bcast(par_v, 3)
    g0 = _bcast(par_v, 4); g1 = _bcast(par_v, 5); g2 = _bcast(par_v, 6)
    e0 = _bcast(par_v, 7); e1 = _bcast(par_v, 8); e2 = _bcast(par_v, 9)
    col0 = jnp.full((16,), 0, jnp.int32)
    col1 = jnp.full((16,), 1, jnp.int32)
    col2 = jnp.full((16,), 2, jnp.int32)
    col3 = jnp.full((16,), 3, jnp.int32)
    zf = jnp.zeros((16,), jnp.float32)

    def grp(g, carry):
        row = g * 16 + lane
        xs = plsc.load_gather(a0, [row, col0]) + plsc.load_gather(a1, [row, col0])
        ys = plsc.load_gather(a0, [row, col1]) + plsc.load_gather(a1, [row, col1])
        zs = plsc.load_gather(a0, [row, col2]) + plsc.load_gather(a1, [row, col2])
        cn = plsc.load_gather(a0, [row, col3]) + plsc.load_gather(a1, [row, col3])
        cnt = jnp.maximum(cn, 1.0)
        fx = xs / cnt + b0
        fy = ys / cnt + b1
        fz = zs / cnt + b2
        mu = (fx + fy + fz) * jnp.float32(1.0 / 3.0)
        ex = fx - mu
        ey = fy - mu
        ez = fz - mu
        var = (ex * ex + ey * ey + ez * ez) * jnp.float32(1.0 / 3.0)
        rs = _rsqrt(var + 1e-5)
        plsc.store_scatter(fo, [row, col0], ex * rs * g0 + e0)
        plsc.store_scatter(fo, [row, col1], ey * rs * g1 + e1)
        plsc.store_scatter(fo, [row, col2], ez * rs * g2 + e2)
        plsc.store_scatter(fo, [row, col3], zf)
        return carry

    lax.fori_loop(0, NODES_W // 16, grp, 0)
    pltpu.sync_copy(fo, feat_hbm.at[pl.ds(nb, NODES_W), :])


@functools.partial(
    pl.kernel,
    compiler_params=_CP,
    out_type=[jax.ShapeDtypeStruct((N_EDGES,), jnp.float32)] * 3,
    mesh=_MESH,
    scratch_types=[
        pltpu.VMEM((3, NCH, CH), jnp.int32),
        pltpu.VMEM((2, BLK, RW), jnp.float32),
        pltpu.VMEM((2, BLK), jnp.float32),
        pltpu.VMEM((2, BLK), jnp.float32),
        pltpu.VMEM((2, BLK), jnp.float32),
        pltpu.SemaphoreType.DMA,
        pltpu.SemaphoreType.DMA,
        pltpu.SemaphoreType.DMA,
        pltpu.SemaphoreType.DMA,
        pltpu.SemaphoreType.DMA,
    ],
)
def _gather_pass(feat_hbm, idx_hbm, ox_hbm, oy_hbm, oz_hbm,
                 idx_v, rows_v, ox_v, oy_v, oz_v,
                 sem_idx, sem_g0, sem_g1, sem_o0, sem_o1):
    c = lax.axis_index("c")
    s = lax.axis_index("s")
    w = c * NS + s
    sem_g = (sem_g0, sem_g1)
    sem_o = (sem_o0, sem_o1)
    lane = lax.iota(jnp.int32, 16)
    col0 = jnp.full((16,), 0, jnp.int32)
    col1 = jnp.full((16,), 1, jnp.int32)
    col2 = jnp.full((16,), 2, jnp.int32)

    def idx_desc(blk, tri):
        rb = pl.multiple_of(blk * NCH, 8)
        return pltpu.make_async_copy(idx_hbm.at[pl.ds(rb, NCH), :],
                                     idx_v.at[tri], sem_idx)

    def g_descs(par, tri):
        return [
            pltpu.make_async_copy(feat_hbm.at[idx_v.at[tri, j]],
                                  rows_v.at[par, pl.ds(CH * j, CH), :],
                                  sem_g[par])
            for j in range(NCH)
        ]

    def o_descs(blk, par):
        eb = pl.multiple_of(blk * BLK, 8)
        return [
            pltpu.make_async_copy(ox_v.at[par], ox_hbm.at[pl.ds(eb, BLK)],
                                  sem_o[par]),
            pltpu.make_async_copy(oy_v.at[par], oy_hbm.at[pl.ds(eb, BLK)],
                                  sem_o[par]),
            pltpu.make_async_copy(oz_v.at[par], oz_hbm.at[pl.ds(eb, BLK)],
                                  sem_o[par]),
        ]

    def extract(par):
        def grp(g, carry2):
            o = g * 16
            row = o + lane
            rb = rows_v.at[par]
            ox_v[par, pl.ds(o, 16)] = plsc.load_gather(rb, [row, col0])
            oy_v[par, pl.ds(o, 16)] = plsc.load_gather(rb, [row, col1])
            oz_v[par, pl.ds(o, 16)] = plsc.load_gather(rb, [row, col2])
            return carry2

        lax.fori_loop(0, BLK // 16, grp, 0)

    idx_desc(w, 0).start()

    def sup(t, carry):
        for u in range(6):
            i6 = t * 6 + u
            par, tri = u % 2, u % 3
            blk = i6 * NW + w

            @pl.when(blk < NBLKS)
            def _():
                idx_desc(blk, tri).wait()
                for d in g_descs(par, tri):
                    d.start()

            @pl.when(blk + NW < NBLKS)
            def _():
                idx_desc(blk + NW, (u + 1) % 3).start()

            @pl.when((blk - 3 * NW < NBLKS) & (i6 >= 3))
            def _():
                for d in o_descs(blk - 3 * NW, (u - 1) % 2):
                    d.wait()

            @pl.when((blk - NW < NBLKS) & (i6 >= 1))
            def _():
                for d in g_descs((u - 1) % 2, (u - 1) % 3):
                    d.wait()
                extract((u - 1) % 2)
                for d in o_descs(blk - NW, (u - 1) % 2):
                    d.start()

        return carry

    lax.fori_loop(0, (KMAX + 3 + 5) // 6, sup, 0)


def kernel(relative_pos, edge_index_i, kernel_dirs, W, b, ln_gamma, ln_beta):
    m3 = kernel_dirs.T @ W.T
    pad1 = jnp.zeros((1,), jnp.float32)
    m3_pad = jnp.concatenate([pad1, m3.reshape(9), jnp.zeros((6,), jnp.float32)])
    params = jnp.concatenate(
        [pad1, b, ln_gamma, ln_beta, jnp.zeros((6,), jnp.float32)])

    px = relative_pos[:, 0]
    py = relative_pos[:, 1]
    pz = relative_pos[:, 2]
    idx2d = edge_index_i.reshape(NROW, CH)
    zeros = jnp.zeros((N_PAD, RW), jnp.float32)

    acc = _scatter_pass(px, py, pz, idx2d, zeros, m3_pad)
    feat = _node_pass(acc, params)
    ox, oy, oz = _gather_pass(feat, idx2d)
    return jnp.stack([ox, oy, oz], axis=1)


# speedup vs baseline: 60.2509x; 1.0778x over previous
import functools

import jax
import jax.numpy as jnp
from jax import lax
from jax.experimental import pallas as pl
from jax.experimental.pallas import tpu as pltpu
from jax.experimental.pallas import tpu_sc as plsc

N_NODES = 100000
N_EDGES = 6400000
NC = 2
NS = 16
NW = NC * NS
CH = 128
NCH = 16
BLK = NCH * CH
NBLKS = N_EDGES // BLK
KMAX = -(-NBLKS // NW)
NROW = N_EDGES // CH
N_PAD = NW * 3136
NODES_W = N_PAD // NW
NODES_S = N_PAD // NS
RW = 8


def _rsqrt(x):
    i = lax.bitcast_convert_type(x, jnp.int32)
    i = jnp.int32(0x5F3759DF) - lax.shift_right_arithmetic(i, 1)
    y = lax.bitcast_convert_type(i, jnp.float32)
    y = y * (1.5 - 0.5 * x * y * y)
    y = y * (1.5 - 0.5 * x * y * y)
    return y


def _bcast(vec_ref, k):
    assert k >= 1
    return plsc.load_gather(vec_ref, [jnp.full((16,), k, jnp.int32)])


_MESH = plsc.VectorSubcoreMesh(core_axis_name="c", subcore_axis_name="s",
                               num_cores=NC, num_subcores=NS)
_CP = pltpu.CompilerParams(needs_layout_passes=False, use_tc_tiling_on_sc=False)


@functools.partial(
    pl.kernel,
    compiler_params=_CP,
    out_type=jax.ShapeDtypeStruct((NC, N_PAD, RW), jnp.float32),
    mesh=_MESH,
    scratch_types=[
        pltpu.VMEM((2, BLK), jnp.float32),
        pltpu.VMEM((2, BLK), jnp.float32),
        pltpu.VMEM((2, BLK), jnp.float32),
        pltpu.VMEM((3, NCH, CH), jnp.int32),
        pltpu.VMEM((2, BLK, RW), jnp.float32),
        pltpu.VMEM((16,), jnp.float32),
        pltpu.VMEM_SHARED((N_PAD, RW), jnp.float32),
        pltpu.SemaphoreType.DMA,
        pltpu.SemaphoreType.DMA,
        pltpu.SemaphoreType.DMA,
    ],
)
def _scatter_pass(px_hbm, py_hbm, pz_hbm, idx_hbm, zeros_hbm, m3_hbm,
                  acc_hbm, px_v, py_v, pz_v, idx_v, contrib, m_v, acc_sh,
                  sem_in0, sem_in1, sem_st):
    c = lax.axis_index("c")
    s = lax.axis_index("s")
    w = c * NS + s
    sem_in = (sem_in0, sem_in1)

    pltpu.sync_copy(zeros_hbm.at[pl.ds(NODES_S * s, NODES_S), :],
                    acc_sh.at[pl.ds(NODES_S * s, NODES_S), :])
    pltpu.sync_copy(m3_hbm, m_v)
    pltpu.sync_copy(zeros_hbm.at[pl.ds(0, BLK), :], contrib.at[0])
    pltpu.sync_copy(zeros_hbm.at[pl.ds(0, BLK), :], contrib.at[1])
    plsc.subcore_barrier()

    lane = lax.iota(jnp.int32, 16)
    col3i = jnp.full((16,), 3, jnp.int32)
    onesi = jnp.full((16,), 1.0, jnp.float32)

    def initgrp(g, carry2):
        p16 = g * 16 + lane
        plsc.store_scatter(contrib.at[0], [p16, col3i], onesi)
        plsc.store_scatter(contrib.at[1], [p16, col3i], onesi)
        return carry2

    lax.fori_loop(0, BLK // 16, initgrp, 0)
    m00 = _bcast(m_v, 1); m01 = _bcast(m_v, 2); m02 = _bcast(m_v, 3)
    m10 = _bcast(m_v, 4); m11 = _bcast(m_v, 5); m12 = _bcast(m_v, 6)
    m20 = _bcast(m_v, 7); m21 = _bcast(m_v, 8); m22 = _bcast(m_v, 9)
    col0 = jnp.full((16,), 0, jnp.int32)
    col1 = jnp.full((16,), 1, jnp.int32)
    col2 = jnp.full((16,), 2, jnp.int32)
    col3 = jnp.full((16,), 3, jnp.int32)
    ones = jnp.full((16,), 1.0, jnp.float32)

    def in_descs(blk, par, tri, sem):
        eb = pl.multiple_of(blk * BLK, 8)
        rb = pl.multiple_of(blk * NCH, 8)
        return [
            pltpu.make_async_copy(px_hbm.at[pl.ds(eb, BLK)], px_v.at[par], sem),
            pltpu.make_async_copy(py_hbm.at[pl.ds(eb, BLK)], py_v.at[par], sem),
            pltpu.make_async_copy(pz_hbm.at[pl.ds(eb, BLK)], pz_v.at[par], sem),
            pltpu.make_async_copy(idx_hbm.at[pl.ds(rb, NCH), :],
                                  idx_v.at[tri], sem),
        ]

    def st_descs(par, tri):
        return [
            pltpu.make_async_copy(contrib.at[par, pl.ds(CH * j, CH), :],
                                  acc_sh.at[idx_v.at[tri, j]], sem_st)
            for j in range(NCH)
        ]

    def compute(par):
        cb = contrib.at[par]

        def one(o):
            px = px_v[par, pl.ds(o, 16)]
            py = py_v[par, pl.ds(o, 16)]
            pz = pz_v[par, pl.ds(o, 16)]
            n2 = px * px + py * py + pz * pz
            r = _rsqrt(n2)
            inv = r - jnp.float32(1e-8) * (r * r)
            dx = px * inv
            dy = py * inv
            dz = pz * inv
            cx = dx * m00 + dy * m10 + dz * m20
            cy = dx * m01 + dy * m11 + dz * m21
            cz = dx * m02 + dy * m12 + dz * m22
            p16 = o + lane
            plsc.store_scatter(cb, [p16, col0], cx)
            plsc.store_scatter(cb, [p16, col1], cy)
            plsc.store_scatter(cb, [p16, col2], cz)

        def grp(g, carry2):
            o = g * 32
            one(o)
            one(o + 16)
            return carry2

        lax.fori_loop(0, BLK // 32, grp, 0)

    for d in in_descs(w, 0, 0, sem_in[0]):
        d.start()

    def sup(t, carry):
        for u in range(6):
            i6 = t * 6 + u
            par, tri = u % 2, u % 3
            blk = i6 * NW + w
            nblk = blk + NW

            @pl.when(nblk < NBLKS)
            def _():
                for d in in_descs(nblk, (u + 1) % 2, (u + 1) % 3,
                                  sem_in[(u + 1) % 2]):
                    d.start()

            @pl.when(blk < NBLKS)
            def _():
                for d in in_descs(blk, par, tri, sem_in[par]):
                    d.wait()
                compute(par)

            @pl.when((blk - NW < NBLKS) & (i6 >= 1))
            def _():
                for d in st_descs((u - 1) % 2, (u - 1) % 3):
                    d.wait()

            @pl.when(blk < NBLKS)
            def _():
                for d in st_descs(par, tri):
                    d.start(add=True)

        return carry

    lax.fori_loop(0, (KMAX + 1 + 5) // 6, sup, 0)
    plsc.subcore_barrier()
    pltpu.sync_copy(acc_sh.at[pl.ds(NODES_S * s, NODES_S), :],
                    acc_hbm.at[c, pl.ds(NODES_S * s, NODES_S), :])


@functools.partial(
    pl.kernel,
    compiler_params=_CP,
    out_type=jax.ShapeDtypeStruct((N_PAD, RW), jnp.float32),
    mesh=_MESH,
    scratch_types=[
        pltpu.VMEM((NODES_W, RW), jnp.float32),
        pltpu.VMEM((NODES_W, RW), jnp.float32),
        pltpu.VMEM((NODES_W, RW), jnp.float32),
        pltpu.VMEM((16,), jnp.float32),
    ],
)
def _node_pass(acc_hbm, par_hbm, feat_hbm, a0, a1, fo, par_v):
    c = lax.axis_index("c")
    s = lax.axis_index("s")
    w = c * NS + s
    nb = w * NODES_W
    pltpu.sync_copy(acc_hbm.at[0, pl.ds(nb, NODES_W), :], a0)
    pltpu.sync_copy(acc_hbm.at[1, pl.ds(nb, NODES_W), :], a1)
    pltpu.sync_copy(par_hbm, par_v)

    lane = lax.iota(jnp.int32, 16)
    b0 = _bcast(par_v, 1); b1 = _bcast(par_v, 2); b2 = _bcast(par_v, 3)
    g0 = _bcast(par_v, 4); g1 = _bcast(par_v, 5); g2 = _bcast(par_v, 6)
    e0 = _bcast(par_v, 7); e1 = _bcast(par_v, 8); e2 = _bcast(par_v, 9)
    col0 = jnp.full((16,), 0, jnp.int32)
    col1 = jnp.full((16,), 1, jnp.int32)
    col2 = jnp.full((16,), 2, jnp.int32)
    col3 = jnp.full((16,), 3, jnp.int32)
    zf = jnp.zeros((16,), jnp.float32)

    def grp(g, carry):
        row = g * 16 + lane
        xs = plsc.load_gather(a0, [row, col0]) + plsc.load_gather(a1, [row, col0])
        ys = plsc.load_gather(a0, [row, col1]) + plsc.load_gather(a1, [row, col1])
        zs = plsc.load_gather(a0, [row, col2]) + plsc.load_gather(a1, [row, col2])
        cn = plsc.load_gather(a0, [row, col3]) + plsc.load_gather(a1, [row, col3])
        cnt = jnp.maximum(cn, 1.0)
        fx = xs / cnt + b0
        fy = ys / cnt + b1
        fz = zs / cnt + b2
        mu = (fx + fy + fz) * jnp.float32(1.0 / 3.0)
        ex = fx - mu
        ey = fy - mu
        ez = fz - mu
        var = (ex * ex + ey * ey + ez * ez) * jnp.float32(1.0 / 3.0)
        rs = _rsqrt(var + 1e-5)
        plsc.store_scatter(fo, [row, col0], ex * rs * g0 + e0)
        plsc.store_scatter(fo, [row, col1], ey * rs * g1 + e1)
        plsc.store_scatter(fo, [row, col2], ez * rs * g2 + e2)
        plsc.store_scatter(fo, [row, col3], zf)
        return carry

    lax.fori_loop(0, NODES_W // 16, grp, 0)
    pltpu.sync_copy(fo, feat_hbm.at[pl.ds(nb, NODES_W), :])


@functools.partial(
    pl.kernel,
    compiler_params=_CP,
    out_type=[jax.ShapeDtypeStruct((N_EDGES,), jnp.float32)] * 3,
    mesh=_MESH,
    scratch_types=[
        pltpu.VMEM((3, NCH, CH), jnp.int32),
        pltpu.VMEM((2, BLK, RW), jnp.float32),
        pltpu.VMEM((2, BLK), jnp.float32),
        pltpu.VMEM((2, BLK), jnp.float32),
        pltpu.VMEM((2, BLK), jnp.float32),
        pltpu.SemaphoreType.DMA,
        pltpu.SemaphoreType.DMA,
        pltpu.SemaphoreType.DMA,
        pltpu.SemaphoreType.DMA,
        pltpu.SemaphoreType.DMA,
    ],
)
def _gather_pass(feat_hbm, idx_hbm, ox_hbm, oy_hbm, oz_hbm,
                 idx_v, rows_v, ox_v, oy_v, oz_v,
                 sem_idx, sem_g0, sem_g1, sem_o0, sem_o1):
    c = lax.axis_index("c")
    s = lax.axis_index("s")
    w = c * NS + s
    sem_g = (sem_g0, sem_g1)
    sem_o = (sem_o0, sem_o1)
    lane = lax.iota(jnp.int32, 16)
    col0 = jnp.full((16,), 0, jnp.int32)
    col1 = jnp.full((16,), 1, jnp.int32)
    col2 = jnp.full((16,), 2, jnp.int32)

    def idx_desc(blk, tri):
        rb = pl.multiple_of(blk * NCH, 8)
        return pltpu.make_async_copy(idx_hbm.at[pl.ds(rb, NCH), :],
                                     idx_v.at[tri], sem_idx)

    def g_descs(par, tri):
        return [
            pltpu.make_async_copy(feat_hbm.at[idx_v.at[tri, j]],
                                  rows_v.at[par, pl.ds(CH * j, CH), :],
                                  sem_g[par])
            for j in range(NCH)
        ]

    def o_descs(blk, par):
        eb = pl.multiple_of(blk * BLK, 8)
        return [
            pltpu.make_async_copy(ox_v.at[par], ox_hbm.at[pl.ds(eb, BLK)],
                                  sem_o[par]),
            pltpu.make_async_copy(oy_v.at[par], oy_hbm.at[pl.ds(eb, BLK)],
                                  sem_o[par]),
            pltpu.make_async_copy(oz_v.at[par], oz_hbm.at[pl.ds(eb, BLK)],
                                  sem_o[par]),
        ]

    def extract(par):
        def grp(g, carry2):
            o = g * 16
            row = o + lane
            rb = rows_v.at[par]
            ox_v[par, pl.ds(o, 16)] = plsc.load_gather(rb, [row, col0])
            oy_v[par, pl.ds(o, 16)] = plsc.load_gather(rb, [row, col1])
            oz_v[par, pl.ds(o, 16)] = plsc.load_gather(rb, [row, col2])
            return carry2

        lax.fori_loop(0, BLK // 16, grp, 0)

    idx_desc(w, 0).start()

    def sup(t, carry):
        for u in range(6):
            i6 = t * 6 + u
            par, tri = u % 2, u % 3
            blk = i6 * NW + w

            @pl.when(blk < NBLKS)
            def _():
                idx_desc(blk, tri).wait()
                for d in g_descs(par, tri):
                    d.start()

            @pl.when(blk + NW < NBLKS)
            def _():
                idx_desc(blk + NW, (u + 1) % 3).start()

            @pl.when((blk - 3 * NW < NBLKS) & (i6 >= 3))
            def _():
                for d in o_descs(blk - 3 * NW, (u - 1) % 2):
                    d.wait()

            @pl.when((blk - NW < NBLKS) & (i6 >= 1))
            def _():
                for d in g_descs((u - 1) % 2, (u - 1) % 3):
                    d.wait()
                extract((u - 1) % 2)
                for d in o_descs(blk - NW, (u - 1) % 2):
                    d.start()

        return carry

    lax.fori_loop(0, (KMAX + 3 + 5) // 6, sup, 0)


def kernel(relative_pos, edge_index_i, kernel_dirs, W, b, ln_gamma, ln_beta):
    m3 = kernel_dirs.T @ W.T
    pad1 = jnp.zeros((1,), jnp.float32)
    m3_pad = jnp.concatenate([pad1, m3.reshape(9), jnp.zeros((6,), jnp.float32)])
    params = jnp.concatenate(
        [pad1, b, ln_gamma, ln_beta, jnp.zeros((6,), jnp.float32)])

    px = relative_pos[:, 0]
    py = relative_pos[:, 1]
    pz = relative_pos[:, 2]
    idx2d = edge_index_i.reshape(NROW, CH)
    zeros = jnp.zeros((N_PAD, RW), jnp.float32)

    acc = _scatter_pass(px, py, pz, idx2d, zeros, m3_pad)
    feat = _node_pass(acc, params)
    ox, oy, oz = _gather_pass(feat, idx2d)
    return jnp.stack([ox, oy, oz], axis=1)
